# Initial kernel scaffold; baseline (speedup 1.0000x reference)
import jax
import jax.numpy as jnp
from jax.experimental import pallas as pl


def kernel(xyz, features, W1, b1, W2, b2, W3, b3):
    raise NotImplementedError("write your pallas kernel here")


# speedup vs baseline: 19.5769x; 19.5769x over previous
import jax
import jax.numpy as jnp
from jax.experimental import pallas as pl
from jax.experimental.pallas import tpu as pltpu
from jax.experimental.pallas import tpu_sc as plsc

B = 8
N = 4096
S = 1024
K = 32
CF = 64
R2 = 0.2 * 0.2
D_PAD = 128
TS = 256
T = 128
GW = 128
NIDX = B * S * K


def _fps_body(x_ref, y_ref, z_ref, nx_ref, ny_ref, nz_ref):
    x = x_ref[...]
    y = y_ref[...]
    z = z_ref[...]
    iota = jax.lax.broadcasted_iota(jnp.int32, (B, N), 1).astype(jnp.float32)
    siota = jax.lax.broadcasted_iota(jnp.int32, (B, S), 1)

    def step(i, carry):
        dist, cx, cy, cz, ax, ay, az = carry
        hit = siota == i
        ax = jnp.where(hit, cx, ax)
        ay = jnp.where(hit, cy, ay)
        az = jnp.where(hit, cz, az)
        dx = x - cx
        dy = y - cy
        dz = z - cz
        d = dx * dx + dy * dy + dz * dz
        dist = jnp.minimum(dist, d)
        m = jnp.max(dist, axis=1, keepdims=True)
        far = jnp.min(jnp.where(dist == m, iota, float(N)), axis=1,
                      keepdims=True)
        one = iota == far
        ncx = jnp.sum(jnp.where(one, x, 0.0), axis=1, keepdims=True)
        ncy = jnp.sum(jnp.where(one, y, 0.0), axis=1, keepdims=True)
        ncz = jnp.sum(jnp.where(one, z, 0.0), axis=1, keepdims=True)
        return dist, ncx, ncy, ncz, ax, ay, az

    dist0 = jnp.full((B, N), 1e10, jnp.float32)
    zero_s = jnp.zeros((B, S), jnp.float32)
    carry = jax.lax.fori_loop(
        0, S, step,
        (dist0, x[:, 0:1], y[:, 0:1], z[:, 0:1], zero_s, zero_s, zero_s))
    nx_ref[...] = carry[4]
    ny_ref[...] = carry[5]
    nz_ref[...] = carry[6]


def _fps(px, py, pz):
    out = jax.ShapeDtypeStruct((B, S), jnp.float32)
    return pl.pallas_call(
        _fps_body,
        out_shape=(out, out, out),
    )(px, py, pz)


def _bq_body(px_ref, py_ref, pz_ref, cx_ref, cy_ref, cz_ref, idx_ref):
    b = pl.program_id(0)
    px = px_ref[...].reshape(1, N)
    py = py_ref[...].reshape(1, N)
    pz = pz_ref[...].reshape(1, N)
    lane = jax.lax.broadcasted_iota(jnp.int32, (TS, B), 1)
    sel = lane == b

    def col(ref):
        return jnp.sum(jnp.where(sel, ref[...], 0.0), axis=1, keepdims=True)

    cx = col(cx_ref)
    cy = col(cy_ref)
    cz = col(cz_ref)
    def bf(v):
        return v.astype(jnp.bfloat16).astype(jnp.float32)

    cxb, cyb, czb = bf(cx), bf(cy), bf(cz)
    pxb, pyb, pzb = bf(px), bf(py), bf(pz)
    d = (-2.0 * (cxb * pxb + cyb * pyb + czb * pzb)
         + (cx * cx + cy * cy + cz * cz)
         + (px * px + py * py + pz * pz))
    iota = jax.lax.broadcasted_iota(jnp.int32, (TS, N), 1).astype(jnp.float32)
    val = jnp.where(d <= R2, iota, float(N))
    base = b * N
    first = None
    for k in range(K):
        mk = jnp.min(val, axis=1, keepdims=True)
        if k == 0:
            first = mk
            out_k = mk
        else:
            out_k = jnp.where(mk >= float(N), first, mk)
        idx_ref[:, k:k + 1] = out_k.astype(jnp.int32) + base
        val = jnp.where(iota == mk, float(N), val)


def _ball_query(px, py, pz, cxt, cyt, czt):
    grid = (B, S // TS)
    return pl.pallas_call(
        _bq_body,
        grid=grid,
        in_specs=[
            pl.BlockSpec((1, 1, N), lambda b, s: (b, 0, 0)),
            pl.BlockSpec((1, 1, N), lambda b, s: (b, 0, 0)),
            pl.BlockSpec((1, 1, N), lambda b, s: (b, 0, 0)),
            pl.BlockSpec((TS, B), lambda b, s: (s, 0)),
            pl.BlockSpec((TS, B), lambda b, s: (s, 0)),
            pl.BlockSpec((TS, B), lambda b, s: (s, 0)),
        ],
        out_specs=pl.BlockSpec((TS, K), lambda b, s: (b * (S // TS) + s, 0)),
        out_shape=jax.ShapeDtypeStruct((B * S, K), jnp.int32),
        compiler_params=pltpu.CompilerParams(
            dimension_semantics=("parallel", "parallel")),
    )(px, py, pz, cxt, cyt, czt)


def _sc_gather(table, flat_idx):
    mesh = plsc.VectorSubcoreMesh(core_axis_name="core",
                                  subcore_axis_name="subcore")

    @pl.kernel(out_type=jax.ShapeDtypeStruct((NIDX, D_PAD), jnp.float32),
               mesh=mesh)
    def gather_kernel(x_hbm, i_hbm, o_hbm):
        def body(i_vmem, o_vmem):
            pltpu.sync_copy(x_hbm.at[i_vmem.at[0]], o_vmem)

        pltpu.emit_pipeline(
            body,
            grid=(NIDX // GW,),
            in_specs=[pl.BlockSpec((1, GW), index_map=lambda i: (0, i))],
            out_specs=[pl.BlockSpec((GW, D_PAD), index_map=lambda i: (i, 0))],
            core_axis_name="subcore",
            dimension_semantics=(pltpu.PARALLEL,),
        )(i_hbm, o_hbm)

    return gather_kernel(table, flat_idx)


def _mlp_body(g_ref, c_ref, w1_ref, w1c_ref, b1_ref, w2_ref, b2_ref,
              w3_ref, b3_ref, o_ref):
    g = g_ref[...]
    h1 = jnp.dot(g, w1_ref[...], preferred_element_type=jnp.float32)
    c = c_ref[...].reshape(T, 3)
    w1c = w1c_ref[...]
    corr = (c[:, 0:1] * w1c[0:1, :]
            + c[:, 1:2] * w1c[1:2, :]
            + c[:, 2:3] * w1c[2:3, :])
    h1 = h1.reshape(T, K, CF) - corr[:, None, :] + b1_ref[...][None]
    h1 = jnp.maximum(h1, 0.0).reshape(T * K, CF)
    h2 = jnp.dot(h1, w2_ref[...], preferred_element_type=jnp.float32)
    h2 = jnp.maximum(h2 + b2_ref[...], 0.0)
    h3 = jnp.dot(h2, w3_ref[...], preferred_element_type=jnp.float32)
    h3 = jnp.maximum(h3 + b3_ref[...], 0.0)
    o_ref[...] = jnp.max(h3.reshape(T, K, 128), axis=1)


def _mlp(gathered, new_xyz, w1g, w1c, b1, w2t, b2, w3t, b3):
    grid = (B * S // T,)
    st = S // T
    return pl.pallas_call(
        _mlp_body,
        grid=grid,
        in_specs=[
            pl.BlockSpec((T * K, D_PAD), lambda i: (i, 0)),
            pl.BlockSpec((1, T, 3), lambda i: (i // st, i % st, 0)),
            pl.BlockSpec((D_PAD, CF), lambda i: (0, 0)),
            pl.BlockSpec((3, CF), lambda i: (0, 0)),
            pl.BlockSpec((1, CF), lambda i: (0, 0)),
            pl.BlockSpec((CF, CF), lambda i: (0, 0)),
            pl.BlockSpec((1, CF), lambda i: (0, 0)),
            pl.BlockSpec((CF, 128), lambda i: (0, 0)),
            pl.BlockSpec((1, 128), lambda i: (0, 0)),
        ],
        out_specs=pl.BlockSpec((T, 128), lambda i: (i, 0)),
        out_shape=jax.ShapeDtypeStruct((B * S, 128), jnp.float32),
        compiler_params=pltpu.CompilerParams(
            dimension_semantics=("parallel",)),
    )(gathered, new_xyz, w1g, w1c, b1, w2t, b2, w3t, b3)


def kernel(xyz, features, W1, b1, W2, b2, W3, b3):
    px = xyz[:, :, 0]
    py = xyz[:, :, 1]
    pz = xyz[:, :, 2]

    nx, ny, nz = _fps(px, py, pz)
    new_xyz = jnp.stack([nx, ny, nz], axis=-1)

    idx = _ball_query(px[:, None, :], py[:, None, :], pz[:, None, :],
                      nx.T, ny.T, nz.T)

    feats_t = jnp.transpose(features, (0, 2, 1))
    table = jnp.concatenate(
        [xyz, feats_t,
         jnp.zeros((B, N, D_PAD - 3 - CF), jnp.float32)],
        axis=-1).reshape(B * N, D_PAD)

    gathered = _sc_gather(table, idx.reshape(1, NIDX))

    w1g = jnp.concatenate(
        [W1.T, jnp.zeros((D_PAD - 3 - CF, CF), jnp.float32)], axis=0)
    w1c = W1[:, :3].T

    nf = _mlp(gathered, new_xyz, w1g, w1c, b1[None, :],
              W2.T, b2[None, :], W3.T, b3[None, :])
    new_features = jnp.transpose(nf.reshape(B, S, 128), (0, 2, 1))
    return new_xyz, new_features


# speedup vs baseline: 21.5287x; 1.0997x over previous
import jax
import jax.numpy as jnp
from jax.experimental import pallas as pl
from jax.experimental.pallas import tpu as pltpu
from jax.experimental.pallas import tpu_sc as plsc

B = 8
N = 4096
S = 1024
K = 32
CF = 64
R2 = 0.2 * 0.2
D_PAD = 128
TS = 256
T = 128
GW = 128
NIDX = B * S * K


def _fps_body(x_ref, y_ref, z_ref, nx_ref, ny_ref, nz_ref):
    x = x_ref[...]
    y = y_ref[...]
    z = z_ref[...]
    iota = jax.lax.broadcasted_iota(jnp.int32, (B, N), 1).astype(jnp.float32)
    siota = jax.lax.broadcasted_iota(jnp.int32, (B, S), 1)

    def step(i, carry):
        dist, cx, cy, cz, ax, ay, az = carry
        hit = siota == i
        ax = jnp.where(hit, cx, ax)
        ay = jnp.where(hit, cy, ay)
        az = jnp.where(hit, cz, az)
        dx = x - cx
        dy = y - cy
        dz = z - cz
        d = dx * dx + dy * dy + dz * dz
        dist = jnp.minimum(dist, d)
        m = jnp.max(dist, axis=1, keepdims=True)
        far = jnp.min(jnp.where(dist == m, iota, float(N)), axis=1,
                      keepdims=True)
        one = iota == far
        ncx = jnp.sum(jnp.where(one, x, 0.0), axis=1, keepdims=True)
        ncy = jnp.sum(jnp.where(one, y, 0.0), axis=1, keepdims=True)
        ncz = jnp.sum(jnp.where(one, z, 0.0), axis=1, keepdims=True)
        return dist, ncx, ncy, ncz, ax, ay, az

    dist0 = jnp.full((B, N), 1e10, jnp.float32)
    zero_s = jnp.zeros((B, S), jnp.float32)
    carry = jax.lax.fori_loop(
        0, S, step,
        (dist0, x[:, 0:1], y[:, 0:1], z[:, 0:1], zero_s, zero_s, zero_s))
    nx_ref[...] = carry[4]
    ny_ref[...] = carry[5]
    nz_ref[...] = carry[6]


def _fps(px, py, pz):
    out = jax.ShapeDtypeStruct((B, S), jnp.float32)
    return pl.pallas_call(
        _fps_body,
        out_shape=(out, out, out),
    )(px, py, pz)


def _bq_body(px_ref, py_ref, pz_ref, cx_ref, cy_ref, cz_ref, idx_ref):
    b = pl.program_id(0)
    px = px_ref[...].reshape(1, N)
    py = py_ref[...].reshape(1, N)
    pz = pz_ref[...].reshape(1, N)
    lane = jax.lax.broadcasted_iota(jnp.int32, (TS, B), 1)
    sel = lane == b

    def col(ref):
        return jnp.sum(jnp.where(sel, ref[...], 0.0), axis=1, keepdims=True)

    cx = col(cx_ref)
    cy = col(cy_ref)
    cz = col(cz_ref)
    def bf(v):
        return v.astype(jnp.bfloat16).astype(jnp.float32)

    cxb, cyb, czb = bf(cx), bf(cy), bf(cz)
    pxb, pyb, pzb = bf(px), bf(py), bf(pz)
    d = (-2.0 * (cxb * pxb + cyb * pyb + czb * pzb)
         + (cx * cx + cy * cy + cz * cz)
         + (px * px + py * py + pz * pz))
    iota = jax.lax.broadcasted_iota(jnp.int32, (TS, N), 1).astype(jnp.float32)
    val = jnp.where(d <= R2, iota, float(N))
    base = b * N
    first = None
    for k in range(K):
        mk = jnp.min(val, axis=1, keepdims=True)
        if k == 0:
            first = mk
            out_k = mk
        else:
            out_k = jnp.where(mk >= float(N), first, mk)
        idx_ref[:, k:k + 1] = out_k.astype(jnp.int32) + base
        val = jnp.where(iota == mk, float(N), val)


def _ball_query(px, py, pz, cxt, cyt, czt):
    grid = (B, S // TS)
    return pl.pallas_call(
        _bq_body,
        grid=grid,
        in_specs=[
            pl.BlockSpec((1, 1, N), lambda b, s: (b, 0, 0)),
            pl.BlockSpec((1, 1, N), lambda b, s: (b, 0, 0)),
            pl.BlockSpec((1, 1, N), lambda b, s: (b, 0, 0)),
            pl.BlockSpec((TS, B), lambda b, s: (s, 0)),
            pl.BlockSpec((TS, B), lambda b, s: (s, 0)),
            pl.BlockSpec((TS, B), lambda b, s: (s, 0)),
        ],
        out_specs=pl.BlockSpec((TS, K), lambda b, s: (b * (S // TS) + s, 0)),
        out_shape=jax.ShapeDtypeStruct((B * S, K), jnp.int32),
        compiler_params=pltpu.CompilerParams(
            dimension_semantics=("parallel", "parallel")),
    )(px, py, pz, cxt, cyt, czt)


def _sc_gather(table, flat_idx):
    mesh = plsc.VectorSubcoreMesh(core_axis_name="core",
                                  subcore_axis_name="subcore")

    @pl.kernel(out_type=jax.ShapeDtypeStruct((NIDX, D_PAD), jnp.float32),
               mesh=mesh)
    def gather_kernel(x_hbm, i_hbm, o_hbm):
        def body(i_vmem, o_vmem):
            pltpu.sync_copy(x_hbm.at[i_vmem.at[0]], o_vmem)

        pltpu.emit_pipeline(
            body,
            grid=(NIDX // GW,),
            in_specs=[pl.BlockSpec((1, GW), index_map=lambda i: (0, i))],
            out_specs=[pl.BlockSpec((GW, D_PAD), index_map=lambda i: (i, 0))],
            core_axis_name=("core", "subcore"),
            dimension_semantics=(pltpu.PARALLEL,),
        )(i_hbm, o_hbm)

    return gather_kernel(table, flat_idx)


def _mlp_body(g_ref, c_ref, w1_ref, w1c_ref, b1_ref, w2_ref, b2_ref,
              w3_ref, b3_ref, o_ref):
    g = g_ref[...]
    h1 = jnp.dot(g, w1_ref[...], preferred_element_type=jnp.float32)
    c = c_ref[...].reshape(T, 3)
    w1c = w1c_ref[...]
    corr = (c[:, 0:1] * w1c[0:1, :]
            + c[:, 1:2] * w1c[1:2, :]
            + c[:, 2:3] * w1c[2:3, :])
    h1 = h1.reshape(T, K, CF) - corr[:, None, :] + b1_ref[...][None]
    h1 = jnp.maximum(h1, 0.0).reshape(T * K, CF)
    h2 = jnp.dot(h1, w2_ref[...], preferred_element_type=jnp.float32)
    h2 = jnp.maximum(h2 + b2_ref[...], 0.0)
    h3 = jnp.dot(h2, w3_ref[...], preferred_element_type=jnp.float32)
    h3 = jnp.maximum(h3 + b3_ref[...], 0.0)
    o_ref[...] = jnp.max(h3.reshape(T, K, 128), axis=1)


def _mlp(gathered, new_xyz, w1g, w1c, b1, w2t, b2, w3t, b3):
    grid = (B * S // T,)
    st = S // T
    return pl.pallas_call(
        _mlp_body,
        grid=grid,
        in_specs=[
            pl.BlockSpec((T * K, D_PAD), lambda i: (i, 0)),
            pl.BlockSpec((1, T, 3), lambda i: (i // st, i % st, 0)),
            pl.BlockSpec((D_PAD, CF), lambda i: (0, 0)),
            pl.BlockSpec((3, CF), lambda i: (0, 0)),
            pl.BlockSpec((1, CF), lambda i: (0, 0)),
            pl.BlockSpec((CF, CF), lambda i: (0, 0)),
            pl.BlockSpec((1, CF), lambda i: (0, 0)),
            pl.BlockSpec((CF, 128), lambda i: (0, 0)),
            pl.BlockSpec((1, 128), lambda i: (0, 0)),
        ],
        out_specs=pl.BlockSpec((T, 128), lambda i: (i, 0)),
        out_shape=jax.ShapeDtypeStruct((B * S, 128), jnp.float32),
        compiler_params=pltpu.CompilerParams(
            dimension_semantics=("parallel",)),
    )(gathered, new_xyz, w1g, w1c, b1, w2t, b2, w3t, b3)


def kernel(xyz, features, W1, b1, W2, b2, W3, b3):
    px = xyz[:, :, 0]
    py = xyz[:, :, 1]
    pz = xyz[:, :, 2]

    nx, ny, nz = _fps(px, py, pz)
    new_xyz = jnp.stack([nx, ny, nz], axis=-1)

    idx = _ball_query(px[:, None, :], py[:, None, :], pz[:, None, :],
                      nx.T, ny.T, nz.T)

    feats_t = jnp.transpose(features, (0, 2, 1))
    table = jnp.concatenate(
        [xyz, feats_t,
         jnp.zeros((B, N, D_PAD - 3 - CF), jnp.float32)],
        axis=-1).reshape(B * N, D_PAD)

    gathered = _sc_gather(table, idx.reshape(1, NIDX))

    w1g = jnp.concatenate(
        [W1.T, jnp.zeros((D_PAD - 3 - CF, CF), jnp.float32)], axis=0)
    w1c = W1[:, :3].T

    nf = _mlp(gathered, new_xyz, w1g, w1c, b1[None, :],
              W2.T, b2[None, :], W3.T, b3[None, :])
    new_features = jnp.transpose(nf.reshape(B, S, 128), (0, 2, 1))
    return new_xyz, new_features


# speedup vs baseline: 22.8644x; 1.0620x over previous
import jax
import jax.numpy as jnp
from jax.experimental import pallas as pl
from jax.experimental.pallas import tpu as pltpu
from jax.experimental.pallas import tpu_sc as plsc

B = 8
N = 4096
S = 1024
K = 32
CF = 64
R2 = 0.2 * 0.2
D_PAD = 128
TS = 256
T = 128
GW = 128
NIDX = B * S * K


def _fps_body(x_ref, y_ref, z_ref, nx_ref, ny_ref, nz_ref):
    x = x_ref[...]
    y = y_ref[...]
    z = z_ref[...]
    iota = jax.lax.broadcasted_iota(jnp.int32, (B, N), 1).astype(jnp.float32)
    siota = jax.lax.broadcasted_iota(jnp.int32, (B, S), 1)

    def step(i, carry):
        dist, cx, cy, cz, ax, ay, az = carry
        hit = siota == i
        ax = jnp.where(hit, cx, ax)
        ay = jnp.where(hit, cy, ay)
        az = jnp.where(hit, cz, az)
        dx = x - cx
        dy = y - cy
        dz = z - cz
        d = dx * dx + dy * dy + dz * dz
        dist = jnp.minimum(dist, d)
        m = jnp.max(dist, axis=1, keepdims=True)
        far = jnp.min(jnp.where(dist == m, iota, float(N)), axis=1,
                      keepdims=True)
        one = iota == far
        ncx = jnp.sum(jnp.where(one, x, 0.0), axis=1, keepdims=True)
        ncy = jnp.sum(jnp.where(one, y, 0.0), axis=1, keepdims=True)
        ncz = jnp.sum(jnp.where(one, z, 0.0), axis=1, keepdims=True)
        return dist, ncx, ncy, ncz, ax, ay, az

    dist0 = jnp.full((B, N), 1e10, jnp.float32)
    zero_s = jnp.zeros((B, S), jnp.float32)
    carry = jax.lax.fori_loop(
        0, S, step,
        (dist0, x[:, 0:1], y[:, 0:1], z[:, 0:1], zero_s, zero_s, zero_s),
        unroll=8)
    nx_ref[...] = carry[4]
    ny_ref[...] = carry[5]
    nz_ref[...] = carry[6]


def _fps(px, py, pz):
    out = jax.ShapeDtypeStruct((B, S), jnp.float32)
    return pl.pallas_call(
        _fps_body,
        out_shape=(out, out, out),
    )(px, py, pz)


def _bq_body(px_ref, py_ref, pz_ref, cx_ref, cy_ref, cz_ref, idx_ref):
    b = pl.program_id(0)
    px = px_ref[...].reshape(1, N)
    py = py_ref[...].reshape(1, N)
    pz = pz_ref[...].reshape(1, N)
    lane = jax.lax.broadcasted_iota(jnp.int32, (TS, B), 1)
    sel = lane == b

    def col(ref):
        return jnp.sum(jnp.where(sel, ref[...], 0.0), axis=1, keepdims=True)

    cx = col(cx_ref)
    cy = col(cy_ref)
    cz = col(cz_ref)
    def bf(v):
        return v.astype(jnp.bfloat16).astype(jnp.float32)

    cxb, cyb, czb = bf(cx), bf(cy), bf(cz)
    pxb, pyb, pzb = bf(px), bf(py), bf(pz)
    d = (-2.0 * (cxb * pxb + cyb * pyb + czb * pzb)
         + (cx * cx + cy * cy + cz * cz)
         + (px * px + py * py + pz * pz))
    iota = jax.lax.broadcasted_iota(jnp.int32, (TS, N), 1).astype(jnp.float32)
    val = jnp.where(d <= R2, iota, float(N))
    base = b * N
    first = None
    for k in range(K):
        mk = jnp.min(val, axis=1, keepdims=True)
        if k == 0:
            first = mk
            out_k = mk
        else:
            out_k = jnp.where(mk >= float(N), first, mk)
        idx_ref[:, k:k + 1] = out_k.astype(jnp.int32) + base
        val = jnp.where(iota == mk, float(N), val)


def _ball_query(px, py, pz, cxt, cyt, czt):
    grid = (B, S // TS)
    return pl.pallas_call(
        _bq_body,
        grid=grid,
        in_specs=[
            pl.BlockSpec((1, 1, N), lambda b, s: (b, 0, 0)),
            pl.BlockSpec((1, 1, N), lambda b, s: (b, 0, 0)),
            pl.BlockSpec((1, 1, N), lambda b, s: (b, 0, 0)),
            pl.BlockSpec((TS, B), lambda b, s: (s, 0)),
            pl.BlockSpec((TS, B), lambda b, s: (s, 0)),
            pl.BlockSpec((TS, B), lambda b, s: (s, 0)),
        ],
        out_specs=pl.BlockSpec((TS, K), lambda b, s: (b * (S // TS) + s, 0)),
        out_shape=jax.ShapeDtypeStruct((B * S, K), jnp.int32),
        compiler_params=pltpu.CompilerParams(
            dimension_semantics=("parallel", "parallel")),
    )(px, py, pz, cxt, cyt, czt)


def _sc_gather(table, flat_idx):
    mesh = plsc.VectorSubcoreMesh(core_axis_name="core",
                                  subcore_axis_name="subcore")

    @pl.kernel(out_type=jax.ShapeDtypeStruct((NIDX, D_PAD), jnp.float32),
               mesh=mesh)
    def gather_kernel(x_hbm, i_hbm, o_hbm):
        def body(i_vmem, o_vmem):
            pltpu.sync_copy(x_hbm.at[i_vmem.at[0]], o_vmem)

        pltpu.emit_pipeline(
            body,
            grid=(NIDX // GW,),
            in_specs=[pl.BlockSpec((1, GW), index_map=lambda i: (0, i))],
            out_specs=[pl.BlockSpec((GW, D_PAD), index_map=lambda i: (i, 0))],
            core_axis_name=("core", "subcore"),
            dimension_semantics=(pltpu.PARALLEL,),
        )(i_hbm, o_hbm)

    return gather_kernel(table, flat_idx)


def _mlp_body(g_ref, c_ref, w1_ref, w1c_ref, b1_ref, w2_ref, b2_ref,
              w3_ref, b3_ref, o_ref):
    g = g_ref[...]
    h1 = jnp.dot(g, w1_ref[...], preferred_element_type=jnp.float32)
    c = c_ref[...].reshape(T, 3)
    w1c = w1c_ref[...]
    corr = (c[:, 0:1] * w1c[0:1, :]
            + c[:, 1:2] * w1c[1:2, :]
            + c[:, 2:3] * w1c[2:3, :])
    h1 = h1.reshape(T, K, CF) - corr[:, None, :] + b1_ref[...][None]
    h1 = jnp.maximum(h1, 0.0).reshape(T * K, CF)
    h2 = jnp.dot(h1, w2_ref[...], preferred_element_type=jnp.float32)
    h2 = jnp.maximum(h2 + b2_ref[...], 0.0)
    h3 = jnp.dot(h2, w3_ref[...], preferred_element_type=jnp.float32)
    h3 = jnp.maximum(h3 + b3_ref[...], 0.0)
    o_ref[...] = jnp.max(h3.reshape(T, K, 128), axis=1)


def _mlp(gathered, new_xyz, w1g, w1c, b1, w2t, b2, w3t, b3):
    grid = (B * S // T,)
    st = S // T
    return pl.pallas_call(
        _mlp_body,
        grid=grid,
        in_specs=[
            pl.BlockSpec((T * K, D_PAD), lambda i: (i, 0)),
            pl.BlockSpec((1, T, 3), lambda i: (i // st, i % st, 0)),
            pl.BlockSpec((D_PAD, CF), lambda i: (0, 0)),
            pl.BlockSpec((3, CF), lambda i: (0, 0)),
            pl.BlockSpec((1, CF), lambda i: (0, 0)),
            pl.BlockSpec((CF, CF), lambda i: (0, 0)),
            pl.BlockSpec((1, CF), lambda i: (0, 0)),
            pl.BlockSpec((CF, 128), lambda i: (0, 0)),
            pl.BlockSpec((1, 128), lambda i: (0, 0)),
        ],
        out_specs=pl.BlockSpec((T, 128), lambda i: (i, 0)),
        out_shape=jax.ShapeDtypeStruct((B * S, 128), jnp.float32),
        compiler_params=pltpu.CompilerParams(
            dimension_semantics=("parallel",)),
    )(gathered, new_xyz, w1g, w1c, b1, w2t, b2, w3t, b3)


def kernel(xyz, features, W1, b1, W2, b2, W3, b3):
    px = xyz[:, :, 0]
    py = xyz[:, :, 1]
    pz = xyz[:, :, 2]

    nx, ny, nz = _fps(px, py, pz)
    new_xyz = jnp.stack([nx, ny, nz], axis=-1)

    idx = _ball_query(px[:, None, :], py[:, None, :], pz[:, None, :],
                      nx.T, ny.T, nz.T)

    feats_t = jnp.transpose(features, (0, 2, 1))
    table = jnp.concatenate(
        [xyz, feats_t,
         jnp.zeros((B, N, D_PAD - 3 - CF), jnp.float32)],
        axis=-1).reshape(B * N, D_PAD)

    gathered = _sc_gather(table, idx.reshape(1, NIDX))

    w1g = jnp.concatenate(
        [W1.T, jnp.zeros((D_PAD - 3 - CF, CF), jnp.float32)], axis=0)
    w1c = W1[:, :3].T

    nf = _mlp(gathered, new_xyz, w1g, w1c, b1[None, :],
              W2.T, b2[None, :], W3.T, b3[None, :])
    new_features = jnp.transpose(nf.reshape(B, S, 128), (0, 2, 1))
    return new_xyz, new_features


# speedup vs baseline: 23.1759x; 1.0136x over previous
import jax
import jax.numpy as jnp
from jax.experimental import pallas as pl
from jax.experimental.pallas import tpu as pltpu
from jax.experimental.pallas import tpu_sc as plsc

B = 8
N = 4096
S = 1024
K = 32
CF = 64
R2 = 0.2 * 0.2
D_PAD = 128
TS = 256
T = 128
GW = 128
NIDX = B * S * K


def _fps_body(x_ref, y_ref, z_ref, nx_ref, ny_ref, nz_ref):
    x = x_ref[...]
    y = y_ref[...]
    z = z_ref[...]
    iota = jax.lax.broadcasted_iota(jnp.int32, (B, N), 1).astype(jnp.float32)
    siota = jax.lax.broadcasted_iota(jnp.int32, (B, S), 1)

    def step(i, carry):
        dist, cx, cy, cz, ax, ay, az = carry
        hit = siota == i
        ax = jnp.where(hit, cx, ax)
        ay = jnp.where(hit, cy, ay)
        az = jnp.where(hit, cz, az)
        dx = x - cx
        dy = y - cy
        dz = z - cz
        d = dx * dx + dy * dy + dz * dz
        dist = jnp.minimum(dist, d)
        m = jnp.max(dist, axis=1, keepdims=True)
        far = jnp.min(jnp.where(dist == m, iota, float(N)), axis=1,
                      keepdims=True)
        one = iota == far
        ncx = jnp.sum(jnp.where(one, x, 0.0), axis=1, keepdims=True)
        ncy = jnp.sum(jnp.where(one, y, 0.0), axis=1, keepdims=True)
        ncz = jnp.sum(jnp.where(one, z, 0.0), axis=1, keepdims=True)
        return dist, ncx, ncy, ncz, ax, ay, az

    dist0 = jnp.full((B, N), 1e10, jnp.float32)
    zero_s = jnp.zeros((B, S), jnp.float32)
    carry = jax.lax.fori_loop(
        0, S, step,
        (dist0, x[:, 0:1], y[:, 0:1], z[:, 0:1], zero_s, zero_s, zero_s),
        unroll=16)
    nx_ref[...] = carry[4]
    ny_ref[...] = carry[5]
    nz_ref[...] = carry[6]


def _fps(px, py, pz):
    out = jax.ShapeDtypeStruct((B, S), jnp.float32)
    return pl.pallas_call(
        _fps_body,
        out_shape=(out, out, out),
    )(px, py, pz)


def _bq_body(px_ref, py_ref, pz_ref, cx_ref, cy_ref, cz_ref, idx_ref):
    b = pl.program_id(0)
    px = px_ref[...].reshape(1, N)
    py = py_ref[...].reshape(1, N)
    pz = pz_ref[...].reshape(1, N)
    lane = jax.lax.broadcasted_iota(jnp.int32, (TS, B), 1)
    sel = lane == b

    def col(ref):
        return jnp.sum(jnp.where(sel, ref[...], 0.0), axis=1, keepdims=True)

    cx = col(cx_ref)
    cy = col(cy_ref)
    cz = col(cz_ref)
    def bf(v):
        return v.astype(jnp.bfloat16).astype(jnp.float32)

    cxb, cyb, czb = bf(cx), bf(cy), bf(cz)
    pxb, pyb, pzb = bf(px), bf(py), bf(pz)
    d = (-2.0 * (cxb * pxb + cyb * pyb + czb * pzb)
         + (cx * cx + cy * cy + cz * cz)
         + (px * px + py * py + pz * pz))
    iota = jax.lax.broadcasted_iota(jnp.int32, (TS, N), 1).astype(jnp.float32)
    val = jnp.where(d <= R2, iota, float(N))
    base = b * N
    first = None
    for k in range(K):
        mk = jnp.min(val, axis=1, keepdims=True)
        if k == 0:
            first = mk
            out_k = mk
        else:
            out_k = jnp.where(mk >= float(N), first, mk)
        idx_ref[:, k:k + 1] = out_k.astype(jnp.int32) + base
        val = jnp.where(iota == mk, float(N), val)


def _ball_query(px, py, pz, cxt, cyt, czt):
    grid = (B, S // TS)
    return pl.pallas_call(
        _bq_body,
        grid=grid,
        in_specs=[
            pl.BlockSpec((1, 1, N), lambda b, s: (b, 0, 0)),
            pl.BlockSpec((1, 1, N), lambda b, s: (b, 0, 0)),
            pl.BlockSpec((1, 1, N), lambda b, s: (b, 0, 0)),
            pl.BlockSpec((TS, B), lambda b, s: (s, 0)),
            pl.BlockSpec((TS, B), lambda b, s: (s, 0)),
            pl.BlockSpec((TS, B), lambda b, s: (s, 0)),
        ],
        out_specs=pl.BlockSpec((TS, K), lambda b, s: (b * (S // TS) + s, 0)),
        out_shape=jax.ShapeDtypeStruct((B * S, K), jnp.int32),
        compiler_params=pltpu.CompilerParams(
            dimension_semantics=("parallel", "parallel")),
    )(px, py, pz, cxt, cyt, czt)


def _sc_gather(table, flat_idx):
    mesh = plsc.VectorSubcoreMesh(core_axis_name="core",
                                  subcore_axis_name="subcore")

    @pl.kernel(out_type=jax.ShapeDtypeStruct((NIDX, D_PAD), jnp.float32),
               mesh=mesh)
    def gather_kernel(x_hbm, i_hbm, o_hbm):
        def body(i_vmem, o_vmem):
            pltpu.sync_copy(x_hbm.at[i_vmem.at[0]], o_vmem)

        pltpu.emit_pipeline(
            body,
            grid=(NIDX // GW,),
            in_specs=[pl.BlockSpec((1, GW), index_map=lambda i: (0, i))],
            out_specs=[pl.BlockSpec((GW, D_PAD), index_map=lambda i: (i, 0))],
            core_axis_name=("core", "subcore"),
            dimension_semantics=(pltpu.PARALLEL,),
        )(i_hbm, o_hbm)

    return gather_kernel(table, flat_idx)


def _mlp_body(g_ref, c_ref, w1_ref, w1c_ref, b1_ref, w2_ref, b2_ref,
              w3_ref, b3_ref, o_ref):
    g = g_ref[...]
    h1 = jnp.dot(g, w1_ref[...], preferred_element_type=jnp.float32)
    c = c_ref[...].reshape(T, 3)
    w1c = w1c_ref[...]
    corr = (c[:, 0:1] * w1c[0:1, :]
            + c[:, 1:2] * w1c[1:2, :]
            + c[:, 2:3] * w1c[2:3, :])
    h1 = h1.reshape(T, K, CF) - corr[:, None, :] + b1_ref[...][None]
    h1 = jnp.maximum(h1, 0.0).reshape(T * K, CF)
    h2 = jnp.dot(h1, w2_ref[...], preferred_element_type=jnp.float32)
    h2 = jnp.maximum(h2 + b2_ref[...], 0.0)
    h3 = jnp.dot(h2, w3_ref[...], preferred_element_type=jnp.float32)
    h3 = jnp.maximum(h3 + b3_ref[...], 0.0)
    o_ref[...] = jnp.max(h3.reshape(T, K, 128), axis=1)


def _mlp(gathered, new_xyz, w1g, w1c, b1, w2t, b2, w3t, b3):
    grid = (B * S // T,)
    st = S // T
    return pl.pallas_call(
        _mlp_body,
        grid=grid,
        in_specs=[
            pl.BlockSpec((T * K, D_PAD), lambda i: (i, 0)),
            pl.BlockSpec((1, T, 3), lambda i: (i // st, i % st, 0)),
            pl.BlockSpec((D_PAD, CF), lambda i: (0, 0)),
            pl.BlockSpec((3, CF), lambda i: (0, 0)),
            pl.BlockSpec((1, CF), lambda i: (0, 0)),
            pl.BlockSpec((CF, CF), lambda i: (0, 0)),
            pl.BlockSpec((1, CF), lambda i: (0, 0)),
            pl.BlockSpec((CF, 128), lambda i: (0, 0)),
            pl.BlockSpec((1, 128), lambda i: (0, 0)),
        ],
        out_specs=pl.BlockSpec((T, 128), lambda i: (i, 0)),
        out_shape=jax.ShapeDtypeStruct((B * S, 128), jnp.float32),
        compiler_params=pltpu.CompilerParams(
            dimension_semantics=("parallel",)),
    )(gathered, new_xyz, w1g, w1c, b1, w2t, b2, w3t, b3)


def kernel(xyz, features, W1, b1, W2, b2, W3, b3):
    px = xyz[:, :, 0]
    py = xyz[:, :, 1]
    pz = xyz[:, :, 2]

    nx, ny, nz = _fps(px, py, pz)
    new_xyz = jnp.stack([nx, ny, nz], axis=-1)

    idx = _ball_query(px[:, None, :], py[:, None, :], pz[:, None, :],
                      nx.T, ny.T, nz.T)

    feats_t = jnp.transpose(features, (0, 2, 1))
    table = jnp.concatenate(
        [xyz, feats_t,
         jnp.zeros((B, N, D_PAD - 3 - CF), jnp.float32)],
        axis=-1).reshape(B * N, D_PAD)

    gathered = _sc_gather(table, idx.reshape(1, NIDX))

    w1g = jnp.concatenate(
        [W1.T, jnp.zeros((D_PAD - 3 - CF, CF), jnp.float32)], axis=0)
    w1c = W1[:, :3].T

    nf = _mlp(gathered, new_xyz, w1g, w1c, b1[None, :],
              W2.T, b2[None, :], W3.T, b3[None, :])
    new_features = jnp.transpose(nf.reshape(B, S, 128), (0, 2, 1))
    return new_xyz, new_features


# speedup vs baseline: 23.2160x; 1.0017x over previous
import jax
import jax.numpy as jnp
from jax.experimental import pallas as pl
from jax.experimental.pallas import tpu as pltpu
from jax.experimental.pallas import tpu_sc as plsc

B = 8
N = 4096
S = 1024
K = 32
CF = 64
R2 = 0.2 * 0.2
D_PAD = 128
TS = 256
T = 128
GW = 128
NIDX = B * S * K


def _fps_body(x_ref, y_ref, z_ref, nx_ref, ny_ref, nz_ref):
    x = x_ref[...]
    y = y_ref[...]
    z = z_ref[...]
    iota = jax.lax.broadcasted_iota(jnp.int32, (B, N), 1).astype(jnp.float32)
    siota = jax.lax.broadcasted_iota(jnp.int32, (B, S), 1)

    def step(i, carry):
        dist, cx, cy, cz, ax, ay, az = carry
        hit = siota == i
        ax = jnp.where(hit, cx, ax)
        ay = jnp.where(hit, cy, ay)
        az = jnp.where(hit, cz, az)
        dx = x - cx
        dy = y - cy
        dz = z - cz
        d = dx * dx + dy * dy + dz * dz
        dist = jnp.minimum(dist, d)
        m = jnp.max(dist, axis=1, keepdims=True)
        far = jnp.min(jnp.where(dist == m, iota, float(N)), axis=1,
                      keepdims=True)
        one = iota == far
        ncx = jnp.sum(jnp.where(one, x, 0.0), axis=1, keepdims=True)
        ncy = jnp.sum(jnp.where(one, y, 0.0), axis=1, keepdims=True)
        ncz = jnp.sum(jnp.where(one, z, 0.0), axis=1, keepdims=True)
        return dist, ncx, ncy, ncz, ax, ay, az

    dist0 = jnp.full((B, N), 1e10, jnp.float32)
    zero_s = jnp.zeros((B, S), jnp.float32)
    carry = jax.lax.fori_loop(
        0, S, step,
        (dist0, x[:, 0:1], y[:, 0:1], z[:, 0:1], zero_s, zero_s, zero_s),
        unroll=16)
    nx_ref[...] = carry[4]
    ny_ref[...] = carry[5]
    nz_ref[...] = carry[6]


def _fps(px, py, pz):
    out = jax.ShapeDtypeStruct((B, S), jnp.float32)
    return pl.pallas_call(
        _fps_body,
        out_shape=(out, out, out),
    )(px, py, pz)


def _bq_body(px_ref, py_ref, pz_ref, cx_ref, cy_ref, cz_ref, idx_ref):
    b = pl.program_id(0)
    px = px_ref[...].reshape(1, N)
    py = py_ref[...].reshape(1, N)
    pz = pz_ref[...].reshape(1, N)
    lane = jax.lax.broadcasted_iota(jnp.int32, (TS, B), 1)
    sel = lane == b

    def col(ref):
        return jnp.sum(jnp.where(sel, ref[...], 0.0), axis=1, keepdims=True)

    cx = col(cx_ref)
    cy = col(cy_ref)
    cz = col(cz_ref)
    def bf(v):
        return v.astype(jnp.bfloat16).astype(jnp.float32)

    cxb, cyb, czb = bf(cx), bf(cy), bf(cz)
    pxb, pyb, pzb = bf(px), bf(py), bf(pz)
    d = (-2.0 * (cxb * pxb + cyb * pyb + czb * pzb)
         + (cx * cx + cy * cy + cz * cz)
         + (px * px + py * py + pz * pz))
    iota = jax.lax.broadcasted_iota(jnp.int32, (TS, N), 1).astype(jnp.float32)
    val = jnp.where(d <= R2, iota, float(N))
    base = b * N
    first = None
    for k in range(K):
        mk = jnp.min(val, axis=1, keepdims=True)
        if k == 0:
            first = mk
            out_k = mk
        else:
            out_k = jnp.where(mk >= float(N), first, mk)
        idx_ref[:, k:k + 1] = out_k.astype(jnp.int32) + base
        val = jnp.where(iota == mk, float(N), val)


def _ball_query(px, py, pz, cxt, cyt, czt):
    grid = (B, S // TS)
    return pl.pallas_call(
        _bq_body,
        grid=grid,
        in_specs=[
            pl.BlockSpec((1, 1, N), lambda b, s: (b, 0, 0)),
            pl.BlockSpec((1, 1, N), lambda b, s: (b, 0, 0)),
            pl.BlockSpec((1, 1, N), lambda b, s: (b, 0, 0)),
            pl.BlockSpec((TS, B), lambda b, s: (s, 0)),
            pl.BlockSpec((TS, B), lambda b, s: (s, 0)),
            pl.BlockSpec((TS, B), lambda b, s: (s, 0)),
        ],
        out_specs=pl.BlockSpec((TS, K), lambda b, s: (b * (S // TS) + s, 0)),
        out_shape=jax.ShapeDtypeStruct((B * S, K), jnp.int32),
        compiler_params=pltpu.CompilerParams(
            dimension_semantics=("parallel", "parallel")),
    )(px, py, pz, cxt, cyt, czt)


def _sc_gather(table, flat_idx):
    mesh = plsc.VectorSubcoreMesh(core_axis_name="core",
                                  subcore_axis_name="subcore")

    @pl.kernel(out_type=jax.ShapeDtypeStruct((NIDX, D_PAD), jnp.float32),
               mesh=mesh)
    def gather_kernel(x_hbm, i_hbm, o_hbm):
        def body(i_vmem, o_vmem):
            pltpu.sync_copy(x_hbm.at[i_vmem.at[0]], o_vmem)

        pltpu.emit_pipeline(
            body,
            grid=(NIDX // GW,),
            in_specs=[pl.BlockSpec((1, GW), index_map=lambda i: (0, i))],
            out_specs=[pl.BlockSpec((GW, D_PAD), index_map=lambda i: (i, 0))],
            core_axis_name=("core", "subcore"),
            dimension_semantics=(pltpu.PARALLEL,),
        )(i_hbm, o_hbm)

    return gather_kernel(table, flat_idx)


def _mlp_body(g_ref, c_ref, w1_ref, w1c_ref, b1_ref, w2_ref, b2_ref,
              w3_ref, b3_ref, o_ref):
    g = g_ref[...]
    h1 = jnp.dot(g, w1_ref[...], preferred_element_type=jnp.float32)
    c = c_ref[...].reshape(T, 3)
    w1c = w1c_ref[...]
    corr = (c[:, 0:1] * w1c[0:1, :]
            + c[:, 1:2] * w1c[1:2, :]
            + c[:, 2:3] * w1c[2:3, :])
    h1 = h1.reshape(T, K, CF) - corr[:, None, :] + b1_ref[...][None]
    h1 = jnp.maximum(h1, 0.0).reshape(T * K, CF)
    h2 = jnp.dot(h1, w2_ref[...], preferred_element_type=jnp.float32)
    h2 = jnp.maximum(h2 + b2_ref[...], 0.0)
    h3 = jnp.dot(h2, w3_ref[...], preferred_element_type=jnp.float32)
    h3 = jnp.maximum(h3 + b3_ref[...], 0.0)
    o_ref[...] = jnp.max(h3.reshape(T, K, 128), axis=1).T.reshape(1, 128, T)


def _mlp(gathered, new_xyz, w1g, w1c, b1, w2t, b2, w3t, b3):
    grid = (B * S // T,)
    st = S // T
    return pl.pallas_call(
        _mlp_body,
        grid=grid,
        in_specs=[
            pl.BlockSpec((T * K, D_PAD), lambda i: (i, 0)),
            pl.BlockSpec((1, T, 3), lambda i: (i // st, i % st, 0)),
            pl.BlockSpec((D_PAD, CF), lambda i: (0, 0)),
            pl.BlockSpec((3, CF), lambda i: (0, 0)),
            pl.BlockSpec((1, CF), lambda i: (0, 0)),
            pl.BlockSpec((CF, CF), lambda i: (0, 0)),
            pl.BlockSpec((1, CF), lambda i: (0, 0)),
            pl.BlockSpec((CF, 128), lambda i: (0, 0)),
            pl.BlockSpec((1, 128), lambda i: (0, 0)),
        ],
        out_specs=pl.BlockSpec((1, 128, T), lambda i: (i // st, 0, i % st)),
        out_shape=jax.ShapeDtypeStruct((B, 128, S), jnp.float32),
        compiler_params=pltpu.CompilerParams(
            dimension_semantics=("parallel",)),
    )(gathered, new_xyz, w1g, w1c, b1, w2t, b2, w3t, b3)


def kernel(xyz, features, W1, b1, W2, b2, W3, b3):
    px = xyz[:, :, 0]
    py = xyz[:, :, 1]
    pz = xyz[:, :, 2]

    nx, ny, nz = _fps(px, py, pz)
    new_xyz = jnp.stack([nx, ny, nz], axis=-1)

    idx = _ball_query(px[:, None, :], py[:, None, :], pz[:, None, :],
                      nx.T, ny.T, nz.T)

    feats_t = jnp.transpose(features, (0, 2, 1))
    table = jnp.concatenate(
        [xyz, feats_t,
         jnp.zeros((B, N, D_PAD - 3 - CF), jnp.float32)],
        axis=-1).reshape(B * N, D_PAD)

    gathered = _sc_gather(table, idx.reshape(1, NIDX))

    w1g = jnp.concatenate(
        [W1.T, jnp.zeros((D_PAD - 3 - CF, CF), jnp.float32)], axis=0)
    w1c = W1[:, :3].T

    new_features = _mlp(gathered, new_xyz, w1g, w1c, b1[None, :],
                        W2.T, b2[None, :], W3.T, b3[None, :])
    return new_xyz, new_features


# speedup vs baseline: 23.6922x; 1.0205x over previous
import jax
import jax.numpy as jnp
from jax.experimental import pallas as pl
from jax.experimental.pallas import tpu as pltpu
from jax.experimental.pallas import tpu_sc as plsc

B = 8
N = 4096
S = 1024
K = 32
CF = 64
R2 = 0.2 * 0.2
D_PAD = 128
TS = 256
T = 128
GW = 256
NIDX = B * S * K


def _fps_body(x_ref, y_ref, z_ref, nx_ref, ny_ref, nz_ref):
    x = x_ref[...]
    y = y_ref[...]
    z = z_ref[...]
    iota = jax.lax.broadcasted_iota(jnp.int32, (B, N), 1).astype(jnp.float32)
    siota = jax.lax.broadcasted_iota(jnp.int32, (B, S), 1)

    def step(i, carry):
        dist, cx, cy, cz, ax, ay, az = carry
        hit = siota == i
        ax = jnp.where(hit, cx, ax)
        ay = jnp.where(hit, cy, ay)
        az = jnp.where(hit, cz, az)
        dx = x - cx
        dy = y - cy
        dz = z - cz
        d = dx * dx + dy * dy + dz * dz
        dist = jnp.minimum(dist, d)
        m = jnp.max(dist, axis=1, keepdims=True)
        far = jnp.min(jnp.where(dist == m, iota, float(N)), axis=1,
                      keepdims=True)
        one = iota == far
        ncx = jnp.sum(jnp.where(one, x, 0.0), axis=1, keepdims=True)
        ncy = jnp.sum(jnp.where(one, y, 0.0), axis=1, keepdims=True)
        ncz = jnp.sum(jnp.where(one, z, 0.0), axis=1, keepdims=True)
        return dist, ncx, ncy, ncz, ax, ay, az

    dist0 = jnp.full((B, N), 1e10, jnp.float32)
    zero_s = jnp.zeros((B, S), jnp.float32)
    carry = jax.lax.fori_loop(
        0, S, step,
        (dist0, x[:, 0:1], y[:, 0:1], z[:, 0:1], zero_s, zero_s, zero_s),
        unroll=16)
    nx_ref[...] = carry[4]
    ny_ref[...] = carry[5]
    nz_ref[...] = carry[6]


def _fps(px, py, pz):
    out = jax.ShapeDtypeStruct((B, S), jnp.float32)
    return pl.pallas_call(
        _fps_body,
        out_shape=(out, out, out),
    )(px, py, pz)


def _bq_body(px_ref, py_ref, pz_ref, cx_ref, cy_ref, cz_ref, idx_ref):
    b = pl.program_id(0)
    px = px_ref[...].reshape(1, N)
    py = py_ref[...].reshape(1, N)
    pz = pz_ref[...].reshape(1, N)
    lane = jax.lax.broadcasted_iota(jnp.int32, (TS, B), 1)
    sel = lane == b

    def col(ref):
        return jnp.sum(jnp.where(sel, ref[...], 0.0), axis=1, keepdims=True)

    cx = col(cx_ref)
    cy = col(cy_ref)
    cz = col(cz_ref)
    def bf(v):
        return v.astype(jnp.bfloat16).astype(jnp.float32)

    cxb, cyb, czb = bf(cx), bf(cy), bf(cz)
    pxb, pyb, pzb = bf(px), bf(py), bf(pz)
    d = (-2.0 * (cxb * pxb + cyb * pyb + czb * pzb)
         + (cx * cx + cy * cy + cz * cz)
         + (px * px + py * py + pz * pz))
    iota = jax.lax.broadcasted_iota(jnp.int32, (TS, N), 1).astype(jnp.float32)
    val = jnp.where(d <= R2, iota, float(N))
    base = b * N
    first = None
    for k in range(K):
        mk = jnp.min(val, axis=1, keepdims=True)
        if k == 0:
            first = mk
            out_k = mk
        else:
            out_k = jnp.where(mk >= float(N), first, mk)
        idx_ref[:, k:k + 1] = out_k.astype(jnp.int32) + base
        val = jnp.where(iota == mk, float(N), val)


def _ball_query(px, py, pz, cxt, cyt, czt):
    grid = (B, S // TS)
    return pl.pallas_call(
        _bq_body,
        grid=grid,
        in_specs=[
            pl.BlockSpec((1, 1, N), lambda b, s: (b, 0, 0)),
            pl.BlockSpec((1, 1, N), lambda b, s: (b, 0, 0)),
            pl.BlockSpec((1, 1, N), lambda b, s: (b, 0, 0)),
            pl.BlockSpec((TS, B), lambda b, s: (s, 0)),
            pl.BlockSpec((TS, B), lambda b, s: (s, 0)),
            pl.BlockSpec((TS, B), lambda b, s: (s, 0)),
        ],
        out_specs=pl.BlockSpec((TS, K), lambda b, s: (b * (S // TS) + s, 0)),
        out_shape=jax.ShapeDtypeStruct((B * S, K), jnp.int32),
        compiler_params=pltpu.CompilerParams(
            dimension_semantics=("parallel", "parallel")),
    )(px, py, pz, cxt, cyt, czt)


def _sc_gather(table, flat_idx):
    mesh = plsc.VectorSubcoreMesh(core_axis_name="core",
                                  subcore_axis_name="subcore")

    @pl.kernel(out_type=jax.ShapeDtypeStruct((NIDX, D_PAD), jnp.float32),
               mesh=mesh)
    def gather_kernel(x_hbm, i_hbm, o_hbm):
        def body(i_vmem, o_vmem):
            pltpu.sync_copy(x_hbm.at[i_vmem.at[0]], o_vmem)

        pltpu.emit_pipeline(
            body,
            grid=(NIDX // GW,),
            in_specs=[pl.BlockSpec((1, GW), index_map=lambda i: (0, i))],
            out_specs=[pl.BlockSpec((GW, D_PAD), index_map=lambda i: (i, 0))],
            core_axis_name=("core", "subcore"),
            dimension_semantics=(pltpu.PARALLEL,),
        )(i_hbm, o_hbm)

    return gather_kernel(table, flat_idx)


def _mlp_body(g_ref, c_ref, w1_ref, w1c_ref, b1_ref, w2_ref, b2_ref,
              w3_ref, b3_ref, o_ref):
    g = g_ref[...]
    h1 = jnp.dot(g, w1_ref[...], preferred_element_type=jnp.float32)
    c = c_ref[...].reshape(T, 3)
    w1c = w1c_ref[...]
    corr = (c[:, 0:1] * w1c[0:1, :]
            + c[:, 1:2] * w1c[1:2, :]
            + c[:, 2:3] * w1c[2:3, :])
    h1 = h1.reshape(T, K, CF) - corr[:, None, :] + b1_ref[...][None]
    h1 = jnp.maximum(h1, 0.0).reshape(T * K, CF)
    h2 = jnp.dot(h1, w2_ref[...], preferred_element_type=jnp.float32)
    h2 = jnp.maximum(h2 + b2_ref[...], 0.0)
    h3 = jnp.dot(h2, w3_ref[...], preferred_element_type=jnp.float32)
    h3 = jnp.maximum(h3 + b3_ref[...], 0.0)
    o_ref[...] = jnp.max(h3.reshape(T, K, 128), axis=1).T.reshape(1, 128, T)


def _mlp(gathered, new_xyz, w1g, w1c, b1, w2t, b2, w3t, b3):
    grid = (B * S // T,)
    st = S // T
    return pl.pallas_call(
        _mlp_body,
        grid=grid,
        in_specs=[
            pl.BlockSpec((T * K, D_PAD), lambda i: (i, 0)),
            pl.BlockSpec((1, T, 3), lambda i: (i // st, i % st, 0)),
            pl.BlockSpec((D_PAD, CF), lambda i: (0, 0)),
            pl.BlockSpec((3, CF), lambda i: (0, 0)),
            pl.BlockSpec((1, CF), lambda i: (0, 0)),
            pl.BlockSpec((CF, CF), lambda i: (0, 0)),
            pl.BlockSpec((1, CF), lambda i: (0, 0)),
            pl.BlockSpec((CF, 128), lambda i: (0, 0)),
            pl.BlockSpec((1, 128), lambda i: (0, 0)),
        ],
        out_specs=pl.BlockSpec((1, 128, T), lambda i: (i // st, 0, i % st)),
        out_shape=jax.ShapeDtypeStruct((B, 128, S), jnp.float32),
        compiler_params=pltpu.CompilerParams(
            dimension_semantics=("parallel",)),
    )(gathered, new_xyz, w1g, w1c, b1, w2t, b2, w3t, b3)


def kernel(xyz, features, W1, b1, W2, b2, W3, b3):
    px = xyz[:, :, 0]
    py = xyz[:, :, 1]
    pz = xyz[:, :, 2]

    nx, ny, nz = _fps(px, py, pz)
    new_xyz = jnp.stack([nx, ny, nz], axis=-1)

    idx = _ball_query(px[:, None, :], py[:, None, :], pz[:, None, :],
                      nx.T, ny.T, nz.T)

    feats_t = jnp.transpose(features, (0, 2, 1))
    table = jnp.concatenate(
        [xyz, feats_t,
         jnp.zeros((B, N, D_PAD - 3 - CF), jnp.float32)],
        axis=-1).reshape(B * N, D_PAD)

    gathered = _sc_gather(table, idx.reshape(1, NIDX))

    w1g = jnp.concatenate(
        [W1.T, jnp.zeros((D_PAD - 3 - CF, CF), jnp.float32)], axis=0)
    w1c = W1[:, :3].T

    new_features = _mlp(gathered, new_xyz, w1g, w1c, b1[None, :],
                        W2.T, b2[None, :], W3.T, b3[None, :])
    return new_xyz, new_features
